# baseline (speedup 1.0000x reference)
import functools

import jax
import jax.numpy as jnp
from jax import lax
from jax.experimental import pallas as pl
from jax.experimental.pallas import tpu as pltpu
from jax.experimental.pallas import tpu_sc as plsc

_NUM_KEYS = 8192
_KEY_DIM = 32
_BATCH = 16384
_OUT_DIM = 10
_PAD_OUT = 128

_BB = 512


def _argmax_decode_kernel(x_ref, keys_ref, values_ref, w_ref, b_ref,
                          idx_ref, dec_ref):
  sim = lax.dot_general(x_ref[...], keys_ref[...], (((1,), (1,)), ((), ())),
                        preferred_element_type=jnp.float32)
  idx_ref[...] = jnp.argmax(sim, axis=1).astype(jnp.int32)

  @pl.when(pl.program_id(0) == 0)
  def _():
    dec = lax.dot_general(values_ref[...], w_ref[...],
                          (((1,), (1,)), ((), ())),
                          preferred_element_type=jnp.float32)
    dec = dec + b_ref[...]
    pad = jnp.zeros((_NUM_KEYS, _PAD_OUT - _OUT_DIM), jnp.float32)
    dec_ref[...] = jnp.concatenate([dec, pad], axis=1)


def _argmax_and_decode(x, keys, values, w, b):
  grid = _BATCH // _BB
  return pl.pallas_call(
      _argmax_decode_kernel,
      grid=(grid,),
      in_specs=[
          pl.BlockSpec((_BB, _KEY_DIM), lambda i: (i, 0)),
          pl.BlockSpec((_NUM_KEYS, _KEY_DIM), lambda i: (0, 0)),
          pl.BlockSpec((_NUM_KEYS, _KEY_DIM), lambda i: (0, 0)),
          pl.BlockSpec((_OUT_DIM, _KEY_DIM), lambda i: (0, 0)),
          pl.BlockSpec((1, _OUT_DIM), lambda i: (0, 0)),
      ],
      out_specs=[
          pl.BlockSpec((_BB,), lambda i: (i,)),
          pl.BlockSpec((_NUM_KEYS, _PAD_OUT), lambda i: (0, 0)),
      ],
      out_shape=[
          jax.ShapeDtypeStruct((_BATCH,), jnp.int32),
          jax.ShapeDtypeStruct((_NUM_KEYS, _PAD_OUT), jnp.float32),
      ],
  )(x, keys, values, w, b)


def _make_sc_gather():
  info = plsc.get_sparse_core_info()
  nc, ns = info.num_cores, info.num_subcores
  nw = nc * ns
  b_per_w = _BATCH // nw
  mesh = plsc.VectorSubcoreMesh(core_axis_name="c", subcore_axis_name="s")

  @functools.partial(
      pl.kernel,
      out_type=jax.ShapeDtypeStruct((_BATCH, _PAD_OUT), jnp.float32),
      mesh=mesh,
      scratch_types=[
          pltpu.VMEM((b_per_w,), jnp.int32),
          pltpu.VMEM((b_per_w, _PAD_OUT), jnp.float32),
          pltpu.SemaphoreType.DMA,
      ],
  )
  def gather(table_hbm, idx_hbm, out_hbm, idx_v, rows_v, sem):
    wid = lax.axis_index("s") * nc + lax.axis_index("c")
    base = wid * b_per_w
    pltpu.sync_copy(idx_hbm.at[pl.ds(base, b_per_w)], idx_v)
    pltpu.async_copy(table_hbm.at[idx_v], rows_v, sem).wait()
    pltpu.sync_copy(rows_v, out_hbm.at[pl.ds(base, b_per_w)])

  return gather


_sc_gather = None


def kernel(x, keys, values, W_dec, b_dec):
  global _sc_gather
  if _sc_gather is None:
    _sc_gather = _make_sc_gather()
  idx, decoded = _argmax_and_decode(x, keys, values, W_dec,
                                    b_dec.reshape(1, _OUT_DIM))
  return _sc_gather(decoded, idx)[:, :_OUT_DIM]


# speedup vs baseline: 1.0148x; 1.0148x over previous
import functools

import jax
import jax.numpy as jnp
from jax import lax
from jax.experimental import pallas as pl
from jax.experimental.pallas import tpu as pltpu
from jax.experimental.pallas import tpu_sc as plsc

_NUM_KEYS = 8192
_KEY_DIM = 32
_BATCH = 16384
_OUT_DIM = 10
_PAD_OUT = 128

_BB = 1024


def _argmax_decode_kernel(x_ref, keys_ref, values_ref, w_ref, b_ref,
                          idx_ref, dec_ref):
  sim = lax.dot_general(x_ref[...], keys_ref[...], (((1,), (1,)), ((), ())),
                        preferred_element_type=jnp.float32)
  idx_ref[...] = jnp.argmax(sim, axis=1).astype(jnp.int32)

  @pl.when(pl.program_id(0) == 0)
  def _():
    dec = lax.dot_general(values_ref[...], w_ref[...],
                          (((1,), (1,)), ((), ())),
                          preferred_element_type=jnp.float32)
    dec = dec + b_ref[...]
    pad = jnp.zeros((_NUM_KEYS, _PAD_OUT - _OUT_DIM), jnp.float32)
    dec_ref[...] = jnp.concatenate([dec, pad], axis=1)


def _argmax_and_decode(x, keys, values, w, b):
  grid = _BATCH // _BB
  return pl.pallas_call(
      _argmax_decode_kernel,
      grid=(grid,),
      in_specs=[
          pl.BlockSpec((_BB, _KEY_DIM), lambda i: (i, 0)),
          pl.BlockSpec((_NUM_KEYS, _KEY_DIM), lambda i: (0, 0)),
          pl.BlockSpec((_NUM_KEYS, _KEY_DIM), lambda i: (0, 0)),
          pl.BlockSpec((_OUT_DIM, _KEY_DIM), lambda i: (0, 0)),
          pl.BlockSpec((1, _OUT_DIM), lambda i: (0, 0)),
      ],
      out_specs=[
          pl.BlockSpec((_BB,), lambda i: (i,)),
          pl.BlockSpec((_NUM_KEYS, _PAD_OUT), lambda i: (0, 0)),
      ],
      out_shape=[
          jax.ShapeDtypeStruct((_BATCH,), jnp.int32),
          jax.ShapeDtypeStruct((_NUM_KEYS, _PAD_OUT), jnp.float32),
      ],
      compiler_params=pltpu.CompilerParams(
          vmem_limit_bytes=100 * 1024 * 1024,
          allow_input_fusion=[True, True, True, True, True]),
  )(x, keys, values, w, b)


def _make_sc_gather():
  info = plsc.get_sparse_core_info()
  nc, ns = info.num_cores, info.num_subcores
  nw = nc * ns
  b_per_w = _BATCH // nw
  mesh = plsc.VectorSubcoreMesh(core_axis_name="c", subcore_axis_name="s")

  @functools.partial(
      pl.kernel,
      out_type=jax.ShapeDtypeStruct((_BATCH, _PAD_OUT), jnp.float32),
      mesh=mesh,
      scratch_types=[
          pltpu.VMEM((b_per_w,), jnp.int32),
          pltpu.VMEM((b_per_w, _PAD_OUT), jnp.float32),
          pltpu.SemaphoreType.DMA,
      ],
  )
  def gather(table_hbm, idx_hbm, out_hbm, idx_v, rows_v, sem):
    wid = lax.axis_index("s") * nc + lax.axis_index("c")
    base = wid * b_per_w
    pltpu.sync_copy(idx_hbm.at[pl.ds(base, b_per_w)], idx_v)
    pltpu.async_copy(table_hbm.at[idx_v], rows_v, sem).wait()
    pltpu.sync_copy(rows_v, out_hbm.at[pl.ds(base, b_per_w)])

  return gather


_sc_gather = None


def kernel(x, keys, values, W_dec, b_dec):
  global _sc_gather
  if _sc_gather is None:
    _sc_gather = _make_sc_gather()
  idx, decoded = _argmax_and_decode(x, keys, values, W_dec,
                                    b_dec.reshape(1, _OUT_DIM))
  return _sc_gather(decoded, idx)[:, :_OUT_DIM]


# speedup vs baseline: 1.0210x; 1.0062x over previous
import functools

import jax
import jax.numpy as jnp
from jax import lax
from jax.experimental import pallas as pl
from jax.experimental.pallas import tpu as pltpu
from jax.experimental.pallas import tpu_sc as plsc

_NUM_KEYS = 8192
_KEY_DIM = 32
_BATCH = 16384
_OUT_DIM = 10
_PAD_OUT = 128

_BB = 1024


def _argmax_decode_kernel(x_ref, keys_ref, values_ref, w_ref, b_ref,
                          idx_ref, dec_ref):
  sim = lax.dot_general(x_ref[...], keys_ref[...], (((1,), (1,)), ((), ())),
                        preferred_element_type=jnp.float32)
  idx_ref[...] = jnp.argmax(sim, axis=1).astype(jnp.int32)

  @pl.when(pl.program_id(0) == 0)
  def _():
    dec = lax.dot_general(values_ref[...], w_ref[...],
                          (((1,), (1,)), ((), ())),
                          preferred_element_type=jnp.float32)
    dec = dec + b_ref[...]
    pad = jnp.zeros((_NUM_KEYS, _PAD_OUT - _OUT_DIM), jnp.float32)
    dec_ref[...] = jnp.concatenate([dec, pad], axis=1)


def _argmax_and_decode(x, keys, values, w, b):
  grid = _BATCH // _BB
  return pl.pallas_call(
      _argmax_decode_kernel,
      grid=(grid,),
      in_specs=[
          pl.BlockSpec((_BB, _KEY_DIM), lambda i: (i, 0)),
          pl.BlockSpec((_NUM_KEYS, _KEY_DIM), lambda i: (0, 0)),
          pl.BlockSpec((_NUM_KEYS, _KEY_DIM), lambda i: (0, 0)),
          pl.BlockSpec((_OUT_DIM, _KEY_DIM), lambda i: (0, 0)),
          pl.BlockSpec((1, _OUT_DIM), lambda i: (0, 0)),
      ],
      out_specs=[
          pl.BlockSpec((_BB,), lambda i: (i,)),
          pl.BlockSpec((_NUM_KEYS, _PAD_OUT), lambda i: (0, 0)),
      ],
      out_shape=[
          jax.ShapeDtypeStruct((_BATCH,), jnp.int32),
          jax.ShapeDtypeStruct((_NUM_KEYS, _PAD_OUT), jnp.float32),
      ],
      compiler_params=pltpu.CompilerParams(
          vmem_limit_bytes=100 * 1024 * 1024,
          allow_input_fusion=[True, True, True, True, True]),
  )(x, keys, values, w, b)


def _make_sc_gather():
  info = plsc.get_sparse_core_info()
  nc, ns = info.num_cores, info.num_subcores
  nw = nc * ns
  b_per_w = _BATCH // nw
  mesh = plsc.VectorSubcoreMesh(core_axis_name="c", subcore_axis_name="s")

  @functools.partial(
      pl.kernel,
      out_type=jax.ShapeDtypeStruct((_BATCH, _PAD_OUT), jnp.float32),
      mesh=mesh,
      scratch_types=[
          pltpu.VMEM((b_per_w,), jnp.int32),
          pltpu.VMEM((4, b_per_w // 4, _PAD_OUT), jnp.float32),
          pltpu.SemaphoreType.DMA,
      ],
  )
  def gather(table_hbm, idx_hbm, out_hbm, idx_v, rows_v, sem):
    wid = lax.axis_index("s") * nc + lax.axis_index("c")
    base = wid * b_per_w
    ch = b_per_w // 4
    pltpu.sync_copy(idx_hbm.at[pl.ds(base, b_per_w)], idx_v)
    copies = [
        pltpu.async_copy(table_hbm.at[idx_v.at[pl.ds(c * ch, ch)]],
                         rows_v.at[c], sem)
        for c in range(4)
    ]
    for c in range(4):
      copies[c].wait()
      pltpu.sync_copy(rows_v.at[c], out_hbm.at[pl.ds(base + c * ch, ch)])

  return gather


_sc_gather = None


def kernel(x, keys, values, W_dec, b_dec):
  global _sc_gather
  if _sc_gather is None:
    _sc_gather = _make_sc_gather()
  idx, decoded = _argmax_and_decode(x, keys, values, W_dec,
                                    b_dec.reshape(1, _OUT_DIM))
  return _sc_gather(decoded, idx)[:, :_OUT_DIM]


# speedup vs baseline: 1.0218x; 1.0008x over previous
import functools

import jax
import jax.numpy as jnp
from jax import lax
from jax.experimental import pallas as pl
from jax.experimental.pallas import tpu as pltpu
from jax.experimental.pallas import tpu_sc as plsc

_NUM_KEYS = 8192
_KEY_DIM = 32
_BATCH = 16384
_OUT_DIM = 10
_PAD_OUT = 128

_BB = 1024


def _argmax_decode_kernel(x_ref, keys_ref, values_ref, w_ref, b_ref,
                          idx_ref, dec_ref):
  sim = lax.dot_general(x_ref[...], keys_ref[...], (((1,), (1,)), ((), ())),
                        preferred_element_type=jnp.float32)
  idx_ref[...] = jnp.argmax(sim, axis=1).astype(jnp.int32)

  @pl.when(pl.program_id(0) == 0)
  def _():
    dec = lax.dot_general(values_ref[...], w_ref[...],
                          (((1,), (1,)), ((), ())),
                          preferred_element_type=jnp.float32)
    dec = dec + b_ref[...]
    pad = jnp.zeros((_NUM_KEYS, _PAD_OUT - _OUT_DIM), jnp.float32)
    dec_ref[...] = jnp.concatenate([dec, pad], axis=1)


def _argmax_and_decode(x, keys, values, w, b):
  grid = _BATCH // _BB
  return pl.pallas_call(
      _argmax_decode_kernel,
      grid=(grid,),
      in_specs=[
          pl.BlockSpec((_BB, _KEY_DIM), lambda i: (i, 0)),
          pl.BlockSpec((_NUM_KEYS, _KEY_DIM), lambda i: (0, 0)),
          pl.BlockSpec((_NUM_KEYS, _KEY_DIM), lambda i: (0, 0)),
          pl.BlockSpec((_OUT_DIM, _KEY_DIM), lambda i: (0, 0)),
          pl.BlockSpec((1, _OUT_DIM), lambda i: (0, 0)),
      ],
      out_specs=[
          pl.BlockSpec((_BB,), lambda i: (i,)),
          pl.BlockSpec((_NUM_KEYS, _PAD_OUT), lambda i: (0, 0)),
      ],
      out_shape=[
          jax.ShapeDtypeStruct((_BATCH,), jnp.int32),
          jax.ShapeDtypeStruct((_NUM_KEYS, _PAD_OUT), jnp.float32),
      ],
      compiler_params=pltpu.CompilerParams(
          vmem_limit_bytes=100 * 1024 * 1024,
          allow_input_fusion=[True, True, True, True, True]),
  )(x, keys, values, w, b)


def _make_sc_gather():
  info = plsc.get_sparse_core_info()
  nc, ns = info.num_cores, info.num_subcores
  nw = nc * ns
  b_per_w = _BATCH // nw
  mesh = plsc.VectorSubcoreMesh(core_axis_name="c", subcore_axis_name="s")

  @functools.partial(
      pl.kernel,
      out_type=jax.ShapeDtypeStruct((_BATCH, _PAD_OUT), jnp.float32),
      mesh=mesh,
      scratch_types=[
          pltpu.VMEM((b_per_w,), jnp.int32),
          pltpu.VMEM((8, b_per_w // 8, _PAD_OUT), jnp.float32),
          pltpu.SemaphoreType.DMA,
      ],
  )
  def gather(table_hbm, idx_hbm, out_hbm, idx_v, rows_v, sem):
    wid = lax.axis_index("s") * nc + lax.axis_index("c")
    base = wid * b_per_w
    ch = b_per_w // 8
    pltpu.sync_copy(idx_hbm.at[pl.ds(base, b_per_w)], idx_v)
    copies = [
        pltpu.async_copy(table_hbm.at[idx_v.at[pl.ds(c * ch, ch)]],
                         rows_v.at[c], sem)
        for c in range(8)
    ]
    for c in range(8):
      copies[c].wait()
      pltpu.sync_copy(rows_v.at[c], out_hbm.at[pl.ds(base + c * ch, ch)])

  return gather


_sc_gather = None


def kernel(x, keys, values, W_dec, b_dec):
  global _sc_gather
  if _sc_gather is None:
    _sc_gather = _make_sc_gather()
  idx, decoded = _argmax_and_decode(x, keys, values, W_dec,
                                    b_dec.reshape(1, _OUT_DIM))
  return _sc_gather(decoded, idx)[:, :_OUT_DIM]
